# baseline (speedup 1.0000x reference)
import numpy as np
import jax
import jax.numpy as jnp
from jax.experimental import pallas as pl

_P = 10000
_J = 16
_DL = 10

_NIDX = np.array(
    [[0, 0, 1], [0, 1, 2], [1, 2, 3], [2, 3, 0], [0, 4, 5], [4, 5, 6],
     [5, 6, 0], [0, 7, 8], [7, 8, 9], [8, 9, 0], [0, 10, 11], [10, 11, 12],
     [11, 12, 0], [0, 13, 14], [13, 14, 15], [14, 15, 0]], dtype=np.int32)
_NMASK = np.array(
    [[0, 1, 1], [1, 1, 1], [1, 1, 1], [0, 0, 0], [0, 1, 1], [1, 1, 1],
     [0, 0, 0], [0, 1, 1], [1, 1, 1], [0, 0, 0], [0, 1, 1], [1, 1, 1],
     [0, 0, 0], [0, 1, 1], [1, 1, 1], [0, 0, 0]], dtype=np.float32)

_SEL = np.zeros((_J, 27, 144), dtype=np.float32)
for _j in range(_J):
    for _n in range(3):
        _src = _NIDX[_j, _n]
        _SEL[_j, 9 * _n:9 * _n + 9, 9 * _src:9 * _src + 9] = (
            _NMASK[_j, _n] * np.eye(9, dtype=np.float32))


def _dh_kernel(x_ref, sel_ref, lat_ref, w1a_ref, w1b_ref, b1_ref, dh_ref):
    x = x_ref[...]
    hp = jnp.sum(sel_ref[...] * x[:, None, :], axis=2)
    lat = lat_ref[...]
    base = jnp.sum(w1b_ref[...] * lat[:, None, :], axis=2) + b1_ref[...]
    hps = jnp.sum(w1a_ref[...] * hp[:, None, :], axis=2)
    dh_ref[...] = (jax.nn.relu(base + hps) - jax.nn.relu(base)) * 0.001


def _main_kernel(w2_ref, m_ref, dh_ref, iv_ref, out_ref):
    w = w2_ref[...]
    dh = dh_ref[...]
    m = jax.nn.relu(m_ref[...])
    q = w * dh[:, None, :]
    r = jnp.sum(q * m, axis=0)
    res = jnp.sum(r, axis=1, keepdims=True)
    out_ref[...] = iv_ref[...] + res


def kernel(input, input_verts, latent_code, mask_param, W1, b1, W2, b2):
    del b2
    x = input.reshape(1, 144)
    sel = jnp.asarray(_SEL)
    w1a = W1[:, :, :27]
    w1b = W1[:, :, 27:]

    dh = pl.pallas_call(
        _dh_kernel,
        out_shape=jax.ShapeDtypeStruct((_J, 64), jnp.float32),
    )(x, sel, latent_code, w1a, w1b, b1)

    m3 = jnp.repeat(mask_param, 3, axis=1)[:, :, None]
    ivr = input_verts.reshape(3 * _P, 1)

    nb = 25
    rows = 3 * _P // nb
    out = pl.pallas_call(
        _main_kernel,
        grid=(nb,),
        in_specs=[
            pl.BlockSpec((_J, rows, 64), lambda i: (0, i, 0)),
            pl.BlockSpec((_J, rows, 1), lambda i: (0, i, 0)),
            pl.BlockSpec((_J, 64), lambda i: (0, 0)),
            pl.BlockSpec((rows, 1), lambda i: (i, 0)),
        ],
        out_specs=pl.BlockSpec((rows, 1), lambda i: (i, 0)),
        out_shape=jax.ShapeDtypeStruct((3 * _P, 1), jnp.float32),
    )(W2, m3, dh, ivr)
    return out.reshape(1, _P, 3)
